# baseline (speedup 1.0000x reference)
import jax
import jax.numpy as jnp
from jax.experimental import pallas as pl
from jax.experimental.pallas import tpu as pltpu

N_HALF = 25000
D = 256
BC = 5000
NCH = N_HALF // BC
NC = 2 * NCH
NBUF = 4
NSUB = 5
BS = BC // NSUB

_SCHED = [(p, j) for j in range(NCH) for p in (0, 1)]


def _mm_kernel(ctx_hbm, smp_hbm, w_ref, b_ref, out_hbm, xbuf, obuf, in_sem, out_sem):
    def in_copy(c, slot):
        src, j = _SCHED[c]
        src_ref = ctx_hbm if src == 0 else smp_hbm
        return pltpu.make_async_copy(
            src_ref.at[pl.ds(j * BC, BC), :], xbuf.at[slot], in_sem.at[slot]
        )

    def out_row(c):
        src, j = _SCHED[c]
        return src * N_HALF + j * BC

    def in_piece_copy(c, slot, k):
        src, j = _SCHED[c]
        src_ref = ctx_hbm if src == 0 else smp_hbm
        return pltpu.make_async_copy(
            src_ref.at[pl.ds(j * BC + k * BS, BS), :],
            xbuf.at[slot, pl.ds(k * BS, BS), :],
            in_sem.at[slot],
        )

    out_copies = {}

    for k in range(NSUB):
        in_piece_copy(0, 0, k).start()
    for s in range(1, NBUF):
        in_copy(s, s).start()

    for c in range(NC):
        slot = c % NBUF
        if c >= NBUF:
            for cp in out_copies.pop(c - NBUF):
                cp.wait()
        if c == 0:
            pieces = []
            for k in range(NSUB):
                in_piece_copy(0, 0, k).wait()
                obuf[0, pl.ds(k * BS, BS), :] = (
                    jnp.dot(
                        xbuf[0, pl.ds(k * BS, BS), :],
                        w_ref[...],
                        preferred_element_type=jnp.float32,
                    )
                    + b_ref[...]
                )
                cp = pltpu.make_async_copy(
                    obuf.at[0, pl.ds(k * BS, BS), :],
                    out_hbm.at[pl.ds(out_row(0) + k * BS, BS), :],
                    out_sem.at[0],
                )
                cp.start()
                pieces.append(cp)
            out_copies[0] = pieces
            if NBUF < NC:
                in_copy(NBUF, 0).start()
            continue
        in_copy(c, slot).wait()
        if c < NC - 1:
            obuf[slot] = (
                jnp.dot(xbuf[slot], w_ref[...], preferred_element_type=jnp.float32)
                + b_ref[...]
            )
            cp = pltpu.make_async_copy(
                obuf.at[slot], out_hbm.at[pl.ds(out_row(c), BC), :], out_sem.at[slot]
            )
            cp.start()
            out_copies[c] = [cp]
        else:
            pieces = []
            for k in range(NSUB):
                obuf[slot, pl.ds(k * BS, BS), :] = (
                    jnp.dot(
                        xbuf[slot, pl.ds(k * BS, BS), :],
                        w_ref[...],
                        preferred_element_type=jnp.float32,
                    )
                    + b_ref[...]
                )
                cp = pltpu.make_async_copy(
                    obuf.at[slot, pl.ds(k * BS, BS), :],
                    out_hbm.at[pl.ds(out_row(c) + k * BS, BS), :],
                    out_sem.at[slot],
                )
                cp.start()
                pieces.append(cp)
            out_copies[c] = pieces
        if c + NBUF < NC:
            in_copy(c + NBUF, slot).start()

    for c in sorted(out_copies):
        for cp in out_copies[c]:
            cp.wait()


def kernel(context, sample, W_proj, b_proj):
    b2d = b_proj.reshape(1, D)
    out = pl.pallas_call(
        _mm_kernel,
        in_specs=[
            pl.BlockSpec(memory_space=pl.ANY),
            pl.BlockSpec(memory_space=pl.ANY),
            pl.BlockSpec(memory_space=pltpu.VMEM),
            pl.BlockSpec(memory_space=pltpu.VMEM),
        ],
        out_specs=pl.BlockSpec(memory_space=pl.ANY),
        out_shape=jax.ShapeDtypeStruct((2 * N_HALF, D), jnp.float32),
        scratch_shapes=[
            pltpu.VMEM((NBUF, BC, D), jnp.float32),
            pltpu.VMEM((NBUF, BC, D), jnp.float32),
            pltpu.SemaphoreType.DMA((NBUF,)),
            pltpu.SemaphoreType.DMA((NBUF,)),
        ],
    )(context, sample, W_proj, b2d)
    return out


# speedup vs baseline: 1.0068x; 1.0068x over previous
import jax
import jax.numpy as jnp
from jax.experimental import pallas as pl
from jax.experimental.pallas import tpu as pltpu

N_HALF = 25000
D = 256
BC = 5000
NCH = N_HALF // BC
NC = 2 * NCH
NBUF = 4
NSUB = 5
BS = BC // NSUB

_SCHED = [(p, j) for j in range(NCH) for p in (0, 1)]


def _mm_kernel(ctx_hbm, smp_hbm, w_ref, b_ref, out_hbm, xbuf, obuf, in_sem, out_sem):
    def in_copy(c, slot):
        src, j = _SCHED[c]
        src_ref = ctx_hbm if src == 0 else smp_hbm
        return pltpu.make_async_copy(
            src_ref.at[pl.ds(j * BC, BC), :], xbuf.at[slot], in_sem.at[slot]
        )

    def out_row(c):
        src, j = _SCHED[c]
        return src * N_HALF + j * BC

    out_copies = {}

    for s in range(NBUF):
        in_copy(s, s).start()

    for c in range(NC):
        slot = c % NBUF
        if c >= NBUF:
            for cp in out_copies.pop(c - NBUF):
                cp.wait()
        in_copy(c, slot).wait()
        if c < NC - 1:
            obuf[slot] = (
                jnp.dot(xbuf[slot], w_ref[...], preferred_element_type=jnp.float32)
                + b_ref[...]
            )
            cp = pltpu.make_async_copy(
                obuf.at[slot], out_hbm.at[pl.ds(out_row(c), BC), :], out_sem.at[slot]
            )
            cp.start()
            out_copies[c] = [cp]
        else:
            pieces = []
            for k in range(NSUB):
                obuf[slot, pl.ds(k * BS, BS), :] = (
                    jnp.dot(
                        xbuf[slot, pl.ds(k * BS, BS), :],
                        w_ref[...],
                        preferred_element_type=jnp.float32,
                    )
                    + b_ref[...]
                )
                cp = pltpu.make_async_copy(
                    obuf.at[slot, pl.ds(k * BS, BS), :],
                    out_hbm.at[pl.ds(out_row(c) + k * BS, BS), :],
                    out_sem.at[slot],
                )
                cp.start()
                pieces.append(cp)
            out_copies[c] = pieces
        if c + NBUF < NC:
            in_copy(c + NBUF, slot).start()

    for c in sorted(out_copies):
        for cp in out_copies[c]:
            cp.wait()


def kernel(context, sample, W_proj, b_proj):
    b2d = b_proj.reshape(1, D)
    out = pl.pallas_call(
        _mm_kernel,
        in_specs=[
            pl.BlockSpec(memory_space=pl.ANY),
            pl.BlockSpec(memory_space=pl.ANY),
            pl.BlockSpec(memory_space=pltpu.VMEM),
            pl.BlockSpec(memory_space=pltpu.VMEM),
        ],
        out_specs=pl.BlockSpec(memory_space=pl.ANY),
        out_shape=jax.ShapeDtypeStruct((2 * N_HALF, D), jnp.float32),
        scratch_shapes=[
            pltpu.VMEM((NBUF, BC, D), jnp.float32),
            pltpu.VMEM((NBUF, BC, D), jnp.float32),
            pltpu.SemaphoreType.DMA((NBUF,)),
            pltpu.SemaphoreType.DMA((NBUF,)),
        ],
    )(context, sample, W_proj, b2d)
    return out
